# baseline (speedup 1.0000x reference)
import functools

import jax
import jax.numpy as jnp
from jax.experimental import pallas as pl
from jax.experimental.pallas import tpu as pltpu


def _lcam_kernel(x_ref, w1_ref, w2_ref, o_ref, *, inv_hw, nb):
    xb = x_ref[...]
    pmax = jnp.max(xb, axis=1)
    pavg = jnp.sum(xb, axis=1) * inv_hw
    p2 = jnp.concatenate([pmax, pavg], axis=0)
    h = jax.lax.dot_general(
        p2, w1_ref[...],
        dimension_numbers=(((1,), (1,)), ((), ())),
        preferred_element_type=jnp.float32)
    h = jnp.maximum(h, 0.0)
    hrow = h[:nb] + h[nb:]
    y = jax.lax.dot_general(
        hrow, w2_ref[...],
        dimension_numbers=(((1,), (0,)), ((), ())),
        preferred_element_type=jnp.float32)
    o_ref[...] = jax.nn.sigmoid(y)[:, None, :]


@jax.jit
def _lcam(x, w1, w2):
    B, C, H, W = x.shape
    C_ = w1.shape[0]
    HW = H * W

    xt = jnp.transpose(x, (0, 2, 3, 1)).reshape(B, HW, C)
    w1m = w1.reshape(C_, C)
    w2m = jnp.transpose(w2.reshape(C, C_))

    nb = 4
    out = pl.pallas_call(
        functools.partial(_lcam_kernel, inv_hw=1.0 / HW, nb=nb),
        out_shape=jax.ShapeDtypeStruct((B, 1, C), jnp.float32),
        grid=(B // nb,),
        in_specs=[
            pl.BlockSpec((nb, HW, C), lambda i: (i, 0, 0)),
            pl.BlockSpec((C_, C), lambda i: (0, 0)),
            pl.BlockSpec((C_, C), lambda i: (0, 0)),
        ],
        out_specs=pl.BlockSpec((nb, 1, C), lambda i: (i, 0, 0)),
        compiler_params=pltpu.CompilerParams(
            dimension_semantics=("parallel",),
            vmem_limit_bytes=64 * 1024 * 1024),
    )(xt, w1m, w2m)

    return out.reshape(B, C, 1, 1).astype(x.dtype)


def kernel(x, w1, w2):
    return _lcam(x, w1, w2)


# speedup vs baseline: 1.1320x; 1.1320x over previous
import functools

import jax
import jax.numpy as jnp
from jax.experimental import pallas as pl
from jax.experimental.pallas import tpu as pltpu


def _lcam_kernel(x_ref, w1_ref, w2_ref, o_ref, *, inv_hw, nb):
    xb = x_ref[...]
    pmax = jnp.max(xb, axis=1)
    pavg = jnp.sum(xb, axis=1) * inv_hw
    p2 = jnp.concatenate([pmax, pavg], axis=0)
    h = jax.lax.dot_general(
        p2, w1_ref[:, 0, :],
        dimension_numbers=(((1,), (1,)), ((), ())),
        preferred_element_type=jnp.float32)
    h = jnp.maximum(h, 0.0)
    hrow = h[:nb] + h[nb:]
    y = jax.lax.dot_general(
        hrow, w2_ref[:, 0, :],
        dimension_numbers=(((1,), (0,)), ((), ())),
        preferred_element_type=jnp.float32)
    o_ref[...] = jax.nn.sigmoid(y)[:, None, :]


@jax.jit
def _lcam(x, w1, w2):
    B, C, H, W = x.shape
    C_ = w1.shape[0]
    HW = H * W

    xt = jnp.transpose(x, (0, 2, 3, 1)).reshape(B, HW, C)
    w1m = w1.reshape(C_, 1, C)
    w2m = jnp.transpose(w2, (1, 2, 3, 0)).reshape(C_, 1, C)

    nb = 4
    out = pl.pallas_call(
        functools.partial(_lcam_kernel, inv_hw=1.0 / HW, nb=nb),
        out_shape=jax.ShapeDtypeStruct((B, 1, C), jnp.float32),
        grid=(B // nb,),
        in_specs=[
            pl.BlockSpec((nb, HW, C), lambda i: (i, 0, 0)),
            pl.BlockSpec((C_, 1, C), lambda i: (0, 0, 0)),
            pl.BlockSpec((C_, 1, C), lambda i: (0, 0, 0)),
        ],
        out_specs=pl.BlockSpec((nb, 1, C), lambda i: (i, 0, 0)),
        compiler_params=pltpu.CompilerParams(
            dimension_semantics=("parallel",),
            vmem_limit_bytes=64 * 1024 * 1024),
    )(xt, w1m, w2m)

    return out.reshape(B, C, 1, 1).astype(x.dtype)


def kernel(x, w1, w2):
    return _lcam(x, w1, w2)
